# baseline (speedup 1.0000x reference)
import functools

import jax
import jax.numpy as jnp
from jax import lax
from jax.experimental import pallas as pl
from jax.experimental.pallas import tpu as pltpu
from jax.experimental.pallas import tpu_sc as plsc

NC = 2
NS = 16
NW = NC * NS
IDX_CHUNK = 128


def _gather_body(n_chunks, uids, cids, utab, itab, u_out, i_out,
                 uidx, cidx, urows, irows, sem):
    wid = lax.axis_index("s") * NC + lax.axis_index("c")
    pltpu.sync_copy(uids.at[wid], uidx)
    pltpu.sync_copy(cids.at[wid], cidx)
    copies = []
    for j in range(n_chunks):
        dst = pl.ds(j * IDX_CHUNK, IDX_CHUNK)
        copies.append(pltpu.async_copy(utab.at[uidx.at[j]], urows.at[dst], sem))
        copies.append(pltpu.async_copy(itab.at[cidx.at[j]], irows.at[dst], sem))
    for cp in copies:
        cp.wait()
    pltpu.sync_copy(urows, u_out.at[wid])
    pltpu.sync_copy(irows, i_out.at[wid])


def _sc_gather(uids, cids, user_table, item_table, rows_per_w, n_chunks, d):
    mesh = plsc.VectorSubcoreMesh(core_axis_name="c", subcore_axis_name="s",
                                  num_cores=NC, num_subcores=NS)
    f = pl.kernel(
        functools.partial(_gather_body, n_chunks),
        out_type=(
            jax.ShapeDtypeStruct((NW, rows_per_w, d), jnp.float32),
            jax.ShapeDtypeStruct((NW, rows_per_w, d), jnp.float32),
        ),
        mesh=mesh,
        scratch_types=[
            pltpu.VMEM((n_chunks, IDX_CHUNK), jnp.int32),
            pltpu.VMEM((n_chunks, IDX_CHUNK), jnp.int32),
            pltpu.VMEM((rows_per_w, d), jnp.float32),
            pltpu.VMEM((rows_per_w, d), jnp.float32),
            pltpu.SemaphoreType.DMA,
        ],
        compiler_params=pltpu.CompilerParams(use_tc_tiling_on_sc=False),
    )
    return f(uids, cids, user_table, item_table)


def _mlp_body(u_ref, i_ref, w1u_ref, w1v_ref, b1_ref, w2_ref, b2_ref,
              w3_ref, b3_ref, out_ref):
    h = (jnp.dot(u_ref[...], w1u_ref[...], preferred_element_type=jnp.float32)
         + jnp.dot(i_ref[...], w1v_ref[...], preferred_element_type=jnp.float32)
         + b1_ref[...])
    h = jnp.maximum(h, 0.0)
    h = jnp.dot(h, w2_ref[...], preferred_element_type=jnp.float32) + b2_ref[...]
    h = jnp.maximum(h, 0.0)
    out_ref[...] = (jnp.sum(h * w3_ref[...], axis=1, keepdims=True)
                    + b3_ref[...])


def _mlp(u_emb, i_emb, W1u, W1v, b1, W2, b2, w3row, b3, blk):
    b, d = u_emb.shape
    h1 = W1u.shape[1]
    h2 = W2.shape[1]
    grid = (b // blk,)
    rep = lambda i: (0, 0)
    return pl.pallas_call(
        _mlp_body,
        grid=grid,
        in_specs=[
            pl.BlockSpec((blk, d), lambda i: (i, 0)),
            pl.BlockSpec((blk, d), lambda i: (i, 0)),
            pl.BlockSpec((d, h1), rep),
            pl.BlockSpec((d, h1), rep),
            pl.BlockSpec((1, h1), rep),
            pl.BlockSpec((h1, h2), rep),
            pl.BlockSpec((1, h2), rep),
            pl.BlockSpec((1, h2), rep),
            pl.BlockSpec((1, 1), rep),
        ],
        out_specs=pl.BlockSpec((blk, 1), lambda i: (i, 0)),
        out_shape=jax.ShapeDtypeStruct((b, 1), jnp.float32),
    )(u_emb, i_emb, W1u, W1v, b1, W2, b2, w3row, b3)


def kernel(user_ids, content_ids, user_table, item_table, W1, b1, W2, b2, W3, b3):
    batch = user_ids.shape[0]
    d = user_table.shape[1]
    rows_per_w = batch // NW
    n_chunks = rows_per_w // IDX_CHUNK

    uids = user_ids.astype(jnp.int32).reshape(NW, n_chunks, IDX_CHUNK)
    cids = content_ids.astype(jnp.int32).reshape(NW, n_chunks, IDX_CHUNK)

    u_emb, i_emb = _sc_gather(uids, cids, user_table, item_table,
                              rows_per_w, n_chunks, d)
    u_emb = u_emb.reshape(batch, d)
    i_emb = i_emb.reshape(batch, d)

    W1u, W1v = W1[:d, :], W1[d:, :]
    out = _mlp(u_emb, i_emb, W1u, W1v, b1.reshape(1, -1), W2,
               b2.reshape(1, -1), W3.reshape(1, -1), b3.reshape(1, 1),
               blk=2048)
    return out
